# baseline (speedup 1.0000x reference)
import functools

import jax
import jax.numpy as jnp
from jax import lax
from jax.experimental import pallas as pl
from jax.experimental.pallas import tpu as pltpu
from jax.experimental.pallas import tpu_sc as plsc


def _make_gather(V, D, B):
    info = plsc.get_sparse_core_info()
    NC, NS = info.num_cores, info.num_subcores
    NW = NC * NS
    assert D == 64 and V % 8 == 0 and B % 16 == 0
    SEG = (V // NW) // 8 * 8
    SH = 7
    R = 1 << SH
    NSLAB = -(-(V - SEG * (NW - 1)) // R)
    NPAIR = -(-NSLAB // 2)
    NSLAB2 = 2 * NPAIR
    CB = 32
    STG = 128
    NIV = B // 16
    mesh = plsc.VectorSubcoreMesh(core_axis_name="c", subcore_axis_name="s")

    @functools.partial(
        pl.kernel,
        mesh=mesh,
        out_type=jax.ShapeDtypeStruct((B + 8, 2 * D), jnp.float32),
        compiler_params=pltpu.CompilerParams(needs_layout_passes=False),
        scratch_types=[
            pltpu.VMEM((B,), jnp.int32),
            pltpu.VMEM((NSLAB2 * CB,), jnp.int32),
            pltpu.VMEM((NSLAB2 * CB,), jnp.int32),
            pltpu.VMEM((R, D), jnp.float32),
            pltpu.VMEM((R, D), jnp.float32),
            pltpu.VMEM((STG, 2 * D), jnp.float32),
            pltpu.VMEM((STG,), jnp.int32),
            pltpu.SMEM((NSLAB2,), jnp.int32),
            pltpu.SMEM((1,), jnp.int32),
            pltpu.SemaphoreType.DMA,
            pltpu.SemaphoreType.DMA,
            pltpu.SemaphoreType.DMA,
        ],
    )
    def k(table_hbm, idx_hbm, out_hbm, idx_v, brow_v, bpos_v, slab_a, slab_b,
          stage_v, opos_v, counts_s, ovf_s, sem_a, sem_b, sem_s):
        wid = lax.axis_index("s") * NC + lax.axis_index("c")
        lo = wid * SEG
        hi = jnp.where(wid == NW - 1, V, lo + SEG)
        dump = jnp.int32(B) + lax.rem(wid, 8)
        lane = lax.iota(jnp.int32, 16)
        mask0 = lane == 0

        pltpu.sync_copy(idx_hbm, idx_v)

        def zero_counts(s, carry):
            counts_s[s] = 0
            return carry

        lax.fori_loop(0, NSLAB2, zero_counts, 0, unroll=False)
        ovf_s[0] = 0

        def collect(kv, carry):
            ivec = idx_v[pl.ds(kv * 16, 16)]
            mb = jnp.logical_and(ivec >= lo, ivec < hi)
            m = jnp.where(mb, 1, 0).astype(jnp.int32)

            for r in range(16):
                @pl.when(m[r] > 0)
                def _():
                    row_g = ivec[r]
                    s = lax.shift_right_logical(row_g - lo, SH)
                    c = counts_s[s]

                    @pl.when(c < CB)
                    def _():
                        flat = s * CB + c
                        plsc.store_scatter(
                            brow_v, [jnp.full((16,), 1, jnp.int32) * flat],
                            jnp.full((16,), 1, jnp.int32) * row_g, mask=mask0)
                        plsc.store_scatter(
                            bpos_v, [jnp.full((16,), 1, jnp.int32) * flat],
                            jnp.full((16,), 1, jnp.int32) * (kv * 16 + r),
                            mask=mask0)
                        counts_s[s] = c + 1

                    @pl.when(c >= CB)
                    def _():
                        ovf_s[0] = 1

            return carry

        lax.fori_loop(0, NIV, collect, 0, unroll=False)

        for q in range(STG // 16):
            opos_v[pl.ds(q * 16, 16)] = jnp.full((16,), 1, jnp.int32) * dump

        def slab_start(s):
            return jnp.minimum(lo + s * R, V - R)

        def fire(s, buf, sem):
            return pltpu.async_copy(
                table_hbm.at[pl.ds(slab_start(s), R), :], buf, sem
            )

        def drain(buf, sem):
            pltpu.make_async_copy(
                table_hbm.at[pl.ds(0, R), :], buf, sem
            ).wait()

        def flush():
            pltpu.async_copy(stage_v, out_hbm.at[opos_v], sem_s).wait()
            for q in range(STG // 16):
                opos_v[pl.ds(q * 16, 16)] = jnp.full((16,), 1, jnp.int32) * dump

        def stage_row(buf, row_local, slot, pos):
            plsc.store_scatter(
                opos_v, [jnp.full((16,), 1, jnp.int32) * slot],
                jnp.full((16,), 1, jnp.int32) * pos, mask=mask0)
            for c in range(D // 16):
                stage_v[slot, pl.ds(c * 16, 16)] = buf[row_local, pl.ds(c * 16, 16)]

        def process(buf, s, sc_in):
            r0 = slab_start(s)
            c_s = counts_s[s]
            need_flush = jnp.logical_and(c_s > 0, sc_in + c_s > STG)

            @pl.when(need_flush)
            def _():
                flush()

            sc0 = jnp.where(need_flush, 0, sc_in)

            @pl.when(c_s > 0)
            def _():
                for v in range(CB // 16):
                    tot = jnp.clip(c_s - v * 16, 0, 16)

                    @pl.when(tot > 0)
                    def _():
                        bvec = brow_v[pl.ds((s * CB + v * 16), 16)]
                        pvec = bpos_v[pl.ds((s * CB + v * 16), 16)]
                        for r in range(16):
                            @pl.when(r < tot)
                            def _():
                                stage_row(buf, bvec[r] - r0,
                                          sc0 + v * 16 + r, pvec[r])

            sc1 = sc0 + c_s

            @pl.when(ovf_s[0] > 0)
            def _():
                def fb(kv, sc):
                    ivec = idx_v[pl.ds(kv * 16, 16)]
                    mb = jnp.logical_and(ivec >= r0, ivec < r0 + R)
                    mseg = jnp.logical_and(ivec >= lo, ivec < hi)
                    mb2 = jnp.logical_and(mb, mseg)
                    m = jnp.where(mb2, 1, 0).astype(jnp.int32)
                    cum = plsc.cumsum(m)
                    tot = cum[15]
                    nf = jnp.logical_and(tot > 0, sc + tot > STG)

                    @pl.when(nf)
                    def _():
                        flush()

                    scb = jnp.where(nf, 0, sc)

                    @pl.when(tot > 0)
                    def _():
                        for r in range(16):
                            @pl.when(m[r] > 0)
                            def _():
                                stage_row(buf, ivec[r] - r0,
                                          scb + cum[r] - 1, kv * 16 + r)

                    return scb + tot

                sc_fb = lax.fori_loop(0, NIV, fb, sc1, unroll=False)

                @pl.when(sc_fb > 0)
                def _():
                    flush()

            sc2 = jnp.where(ovf_s[0] > 0, 0, sc1)
            return sc2

        fire(0, slab_a, sem_a)
        fire(1, slab_b, sem_b)

        def pair(p, sc):
            drain(slab_a, sem_a)
            sc = process(slab_a, 2 * p, sc)
            fire(2 * p + 2, slab_a, sem_a)
            drain(slab_b, sem_b)
            sc = process(slab_b, 2 * p + 1, sc)
            fire(2 * p + 3, slab_b, sem_b)
            return sc

        sc = lax.fori_loop(0, NPAIR, pair, jnp.int32(0), unroll=False)
        drain(slab_a, sem_a)
        drain(slab_b, sem_b)

        @pl.when(sc > 0)
        def _():
            flush()

    def run(full_tensor, idx32):
        out_pad = k(full_tensor, idx32)
        return lax.slice(out_pad, (0, 0), (B, D))

    return run


def kernel(full_tensor, indices):
    V, D = full_tensor.shape
    (B,) = indices.shape
    idx32 = indices.astype(jnp.int32)
    return _make_gather(V, D, B)(full_tensor, idx32)


# speedup vs baseline: 1.9846x; 1.9846x over previous
import functools

import jax
import jax.numpy as jnp
from jax import lax
from jax.experimental import pallas as pl
from jax.experimental.pallas import tpu as pltpu
from jax.experimental.pallas import tpu_sc as plsc


def _make_gather(V, D, B):
    info = plsc.get_sparse_core_info()
    NC, NS = info.num_cores, info.num_subcores
    NW = NC * NS
    assert B % NW == 0 and (B // NW) % 8 == 0
    b_per_w = B // NW
    K = 32
    NSEM = 8
    assert b_per_w % K == 0
    mesh = plsc.VectorSubcoreMesh(core_axis_name="c", subcore_axis_name="s")

    @functools.partial(
        pl.kernel,
        mesh=mesh,
        out_type=jax.ShapeDtypeStruct((B, D), jnp.float32),
        scratch_types=[
            pltpu.VMEM((b_per_w,), jnp.int32),
            pltpu.VMEM((b_per_w, D), jnp.float32),
        ] + [pltpu.SemaphoreType.DMA] * NSEM,
    )
    def k(table_hbm, idx_hbm, out_hbm, idx_v, rows_v, *sems):
        wid = lax.axis_index("s") * NC + lax.axis_index("c")
        base = wid * b_per_w
        pltpu.sync_copy(idx_hbm.at[pl.ds(base, b_per_w)], idx_v)

        def fire(g):
            handles = []
            for v in range(K // 16):
                ivec = idx_v[pl.ds(g * K + v * 16, 16)]
                for r in range(16):
                    dst = g * K + v * 16 + r
                    handles.append(
                        pltpu.async_copy(
                            table_hbm.at[pl.ds(ivec[r], 1), :],
                            rows_v.at[pl.ds(dst, 1), :],
                            sems[dst % NSEM],
                        )
                    )
            return handles

        n_chunks = b_per_w // K
        prev = fire(0)
        for g in range(1, n_chunks):
            cur = fire(g)
            for h in prev:
                h.wait()
            prev = cur
        for h in prev:
            h.wait()
        pltpu.sync_copy(rows_v, out_hbm.at[pl.ds(base, b_per_w)])

    return k


def kernel(full_tensor, indices):
    V, D = full_tensor.shape
    (B,) = indices.shape
    idx32 = indices.astype(jnp.int32)
    return _make_gather(V, D, B)(full_tensor, idx32)
